# Initial kernel scaffold; baseline (speedup 1.0000x reference)
import jax
import jax.numpy as jnp
from jax.experimental import pallas as pl


def kernel(edge_features, neighbors_edge_idxs, W, b):
    raise NotImplementedError("write your pallas kernel here")


# speedup vs baseline: 1.4264x; 1.4264x over previous
import functools

import jax
import jax.numpy as jnp
from jax import lax
from jax.experimental import pallas as pl
from jax.experimental.pallas import tpu as pltpu
from jax.experimental.pallas import tpu_sc as plsc

_B = 100000
_OUT_D = 64
_IN_D = 512


_BLK = 2000


def _transform_body(e_ref, w_ref, b_ref, o_ref):
    o_ref[...] = (
        lax.dot_general(
            e_ref[...],
            w_ref[...],
            (((1,), (1,)), ((), ())),
            preferred_element_type=jnp.float32,
        )
        + b_ref[...]
    )


def _transform(e, w, bias):
    m = e.shape[0]
    return pl.pallas_call(
        _transform_body,
        grid=(m // _BLK,),
        in_specs=[
            pl.BlockSpec((_BLK, _IN_D), lambda i: (i, 0)),
            pl.BlockSpec((_OUT_D, _IN_D), lambda i: (0, 0)),
            pl.BlockSpec((1, _OUT_D), lambda i: (0, 0)),
        ],
        out_specs=pl.BlockSpec((_BLK, _OUT_D), lambda i: (i, 0)),
        out_shape=jax.ShapeDtypeStruct((m, _OUT_D), jnp.float32),
    )(e, w, bias.reshape(1, _OUT_D))


_NW = 32
_CHUNK = 128
_NCHUNK = 25
_BPW = _CHUNK * _NCHUNK
_BPAD = _NW * _BPW


def _gather_body(t_hbm, idx_hbm, out_hbm, idx_v, rows_v, sem):
    wid = lax.axis_index("s") * 2 + lax.axis_index("c")
    base = wid * _BPW
    pltpu.sync_copy(idx_hbm.at[pl.ds(base, _BPW)], idx_v)

    def chunk(c, carry):
        off = c * _CHUNK
        pltpu.async_copy(
            t_hbm.at[idx_v.at[pl.ds(off, _CHUNK)]], rows_v, sem
        ).wait()
        pltpu.sync_copy(rows_v, out_hbm.at[pl.ds(base + off, _CHUNK)])
        return carry

    lax.fori_loop(0, _NCHUNK, chunk, 0)


_gather = pl.kernel(
    _gather_body,
    mesh=plsc.VectorSubcoreMesh(core_axis_name="c", subcore_axis_name="s"),
    out_type=jax.ShapeDtypeStruct((_BPAD, _OUT_D), jnp.float32),
    scratch_types=[
        pltpu.VMEM((_BPW,), jnp.int32),
        pltpu.VMEM((_CHUNK, _OUT_D), jnp.float32),
        pltpu.SemaphoreType.DMA,
    ],
    compiler_params=pltpu.CompilerParams(use_tc_tiling_on_sc=False),
)


def kernel(edge_features, neighbors_edge_idxs, W, b):
    t = _transform(edge_features, W, b)
    idx = jnp.pad(neighbors_edge_idxs.astype(jnp.int32), (0, _BPAD - _B))
    out_pad = _gather(t, idx)
    return out_pad[:_B]
